# Initial kernel scaffold; baseline (speedup 1.0000x reference)
import jax
import jax.numpy as jnp
from jax.experimental import pallas as pl


def kernel(x, senders, receivers, w_self, b_self, w_neigh, zero_edges):
    raise NotImplementedError("write your pallas kernel here")


# speedup vs baseline: 3.5725x; 3.5725x over previous
import jax
import jax.numpy as jnp
from jax import lax
from jax.experimental import pallas as pl
from jax.experimental.pallas import tpu as pltpu
from jax.experimental.pallas import tpu_sc as plsc

NC = 2
NS = 16
NW = NC * NS
BATCH = 128
CH_B = 8
DEG_L = 16
ZCH = 16
ZCH_D = 64


def _sc_agg_body(x_hbm, send_hbm, recv_hbm, zrow_hbm, zdeg_hbm, ones_hbm,
                 sum_out, deg_out,
                 send_v, recv_v, rows_v, ones_v, zrow_v, zdeg_v,
                 sum_sh, deg_sh, sem):
    nb_per_w = send_hbm.shape[0] // NW
    rt = sum_sh.shape[0] // NS
    c = lax.axis_index("c")
    s = lax.axis_index("s")
    wid = s * NC + c

    pltpu.sync_copy(zrow_hbm, zrow_v)
    pltpu.sync_copy(zdeg_hbm, zdeg_v)
    pltpu.sync_copy(ones_hbm, ones_v)

    def zbody(k, carry):
        pltpu.sync_copy(zrow_v, sum_sh.at[pl.ds(s * rt + k * ZCH, ZCH), :])
        return carry
    lax.fori_loop(0, rt // ZCH, zbody, 0)
    for k in range(rt // ZCH_D):
        pltpu.sync_copy(zdeg_v, deg_sh.at[pl.ds(s * rt + k * ZCH_D, ZCH_D), :])
    plsc.subcore_barrier()

    def chunk(g, carry):
        base = wid * nb_per_w + g * CH_B
        pltpu.sync_copy(send_hbm.at[pl.ds(base, CH_B), :], send_v)
        pltpu.sync_copy(recv_hbm.at[pl.ds(base, CH_B), :], recv_v)
        for j in range(CH_B):
            pltpu.async_copy(x_hbm.at[send_v.at[j]], rows_v, sem).wait()
            pltpu.sync_copy(rows_v, sum_sh.at[recv_v.at[j]], add=True)
            pltpu.sync_copy(ones_v, deg_sh.at[recv_v.at[j]], add=True)
        return carry

    lax.fori_loop(0, nb_per_w // CH_B, chunk, 0)
    plsc.subcore_barrier()

    pltpu.sync_copy(sum_sh.at[pl.ds(s * rt, rt), :],
                    sum_out.at[c, pl.ds(s * rt, rt), :])
    pltpu.sync_copy(deg_sh.at[pl.ds(s * rt, rt), :],
                    deg_out.at[c, pl.ds(s * rt, rt), :])


def _sc_aggregate(x, send2d, recv2d, r_rows):
    d = x.shape[1]
    zrow = jnp.zeros((ZCH, d), jnp.float32)
    zdeg = jnp.zeros((ZCH_D, DEG_L), jnp.float32)
    ones = jnp.ones((BATCH, DEG_L), jnp.float32)

    mesh = plsc.VectorSubcoreMesh(core_axis_name="c", subcore_axis_name="s")
    f = pl.kernel(
        _sc_agg_body,
        out_type=(
            jax.ShapeDtypeStruct((NC, r_rows, d), jnp.float32),
            jax.ShapeDtypeStruct((NC, r_rows, DEG_L), jnp.float32),
        ),
        mesh=mesh,
        scratch_types=(
            pltpu.VMEM((CH_B, BATCH), jnp.int32),
            pltpu.VMEM((CH_B, BATCH), jnp.int32),
            pltpu.VMEM((BATCH, d), jnp.float32),
            pltpu.VMEM((BATCH, DEG_L), jnp.float32),
            pltpu.VMEM((ZCH, d), jnp.float32),
            pltpu.VMEM((ZCH_D, DEG_L), jnp.float32),
            pltpu.VMEM_SHARED((r_rows, d), jnp.float32),
            pltpu.VMEM_SHARED((r_rows, DEG_L), jnp.float32),
            pltpu.SemaphoreType.DMA,
        ),
        compiler_params=pltpu.CompilerParams(use_tc_tiling_on_sc=False),
    )
    return f(x, send2d, recv2d, zrow, zdeg, ones)


def _finalize_body(x_ref, sum_ref, deg_ref, wsT_ref, wnT_ref, b_ref,
                   scale_ref, out_ref):
    xb = x_ref[...]
    sb = sum_ref[0] + sum_ref[1]
    db = deg_ref[0, :, 0:1] + deg_ref[1, :, 0:1]
    recip = jnp.where(db > 0, scale_ref[0] / db, 0.0)
    agg = sb * recip
    acc = lax.dot_general(xb, wsT_ref[...], (((1,), (0,)), ((), ())),
                          precision=lax.Precision.HIGHEST,
                          preferred_element_type=jnp.float32)
    acc = acc + lax.dot_general(agg, wnT_ref[...], (((1,), (0,)), ((), ())),
                                precision=lax.Precision.HIGHEST,
                                preferred_element_type=jnp.float32)
    out_ref[...] = acc + b_ref[...]


def _finalize(x, sum_p, deg_p, w_self, b_self, w_neigh, scale):
    n, d = x.shape
    bn = 1000
    grid = (n // bn,)
    return pl.pallas_call(
        _finalize_body,
        grid=grid,
        in_specs=[
            pl.BlockSpec((bn, d), lambda i: (i, 0)),
            pl.BlockSpec((NC, bn, d), lambda i: (0, i, 0)),
            pl.BlockSpec((NC, bn, DEG_L), lambda i: (0, i, 0)),
            pl.BlockSpec((d, d), lambda i: (0, 0)),
            pl.BlockSpec((d, d), lambda i: (0, 0)),
            pl.BlockSpec((1, d), lambda i: (0, 0)),
            pl.BlockSpec(memory_space=pltpu.SMEM),
        ],
        out_specs=pl.BlockSpec((bn, d), lambda i: (i, 0)),
        out_shape=jax.ShapeDtypeStruct((n, d), jnp.float32),
    )(x, sum_p, deg_p, w_self.T, w_neigh.T, b_self.reshape(1, d), scale)


def kernel(x, senders, receivers, w_self, b_self, w_neigh, zero_edges):
    n, d = x.shape
    e = senders.shape[0]
    senders = senders.astype(jnp.int32)
    receivers = receivers.astype(jnp.int32)

    nb_total = -(-(-(-e // BATCH)) // (NW * CH_B)) * NW * CH_B
    pad_e = nb_total * BATCH - e
    rt = -(-(n + 1) // (NW * ZCH_D)) * ZCH_D
    r_rows = NW * rt
    if pad_e:
        senders = jnp.concatenate(
            [senders, jnp.zeros((pad_e,), jnp.int32)])
        receivers = jnp.concatenate(
            [receivers,
             n + (jnp.arange(pad_e, dtype=jnp.int32) % (r_rows - n))])
    send2d = senders.reshape(nb_total, BATCH)
    recv2d = receivers.reshape(nb_total, BATCH)

    sum_p, deg_p = _sc_aggregate(x, send2d, recv2d, r_rows)

    scale = jnp.where(zero_edges, 0.0, 1.0).astype(jnp.float32).reshape(1)
    return _finalize(x, sum_p, deg_p, w_self, b_self, w_neigh, scale)


# speedup vs baseline: 3.9096x; 1.0944x over previous
import jax
import jax.numpy as jnp
from jax import lax
from jax.experimental import pallas as pl
from jax.experimental.pallas import tpu as pltpu
from jax.experimental.pallas import tpu_sc as plsc

NC = 2
NS = 16
NW = NC * NS
BATCH = 128
CH_B = 8
DEG_L = 16
ZCH = 8
ZCH_D = 32


def _sc_agg_body(x_hbm, send_hbm, recv_hbm, zrow_hbm, zdeg_hbm, ones_hbm,
                 sum_out, deg_out,
                 send_v, recv_v, rows0, rows1, ones_v, zrow_v, zdeg_v,
                 sum_sh, deg_sh, gsem0, gsem1, ssem0, ssem1, dsem):
    nb_per_w = send_hbm.shape[0] // NW
    rt = sum_sh.shape[0] // NS
    c = lax.axis_index("c")
    s = lax.axis_index("s")
    wid = s * NC + c

    pltpu.sync_copy(zrow_hbm, zrow_v)
    pltpu.sync_copy(zdeg_hbm, zdeg_v)
    pltpu.sync_copy(ones_hbm, ones_v)

    def zbody(k, carry):
        pltpu.sync_copy(zrow_v, sum_sh.at[pl.ds(s * rt + k * ZCH, ZCH), :])
        return carry
    lax.fori_loop(0, rt // ZCH, zbody, 0)
    for k in range(rt // ZCH_D):
        pltpu.sync_copy(zdeg_v, deg_sh.at[pl.ds(s * rt + k * ZCH_D, ZCH_D), :])
    plsc.subcore_barrier()

    rows = (rows0, rows1)
    gsem = (gsem0, gsem1)

    def chunk(g, carry):
        base = wid * nb_per_w + g * CH_B
        pltpu.sync_copy(send_hbm.at[pl.ds(base, CH_B), :], send_v)
        pltpu.sync_copy(recv_hbm.at[pl.ds(base, CH_B), :], recv_v)
        gd = [None, None]
        sd = [None, None]
        dd = []
        gd[0] = pltpu.async_copy(x_hbm.at[send_v.at[0]], rows0, gsem0)
        for j in range(CH_B):
            p = j & 1
            q = 1 - p
            if j + 1 < CH_B:
                if sd[q] is not None:
                    sd[q].wait()
                gd[q] = pltpu.async_copy(
                    x_hbm.at[send_v.at[j + 1]], rows[q], gsem[q])
            gd[p].wait()
            sd[p] = pltpu.async_copy(
                rows[p], sum_sh.at[recv_v.at[j]], ssem0 if p == 0 else ssem1,
                add=True)
            dd.append(pltpu.async_copy(
                ones_v, deg_sh.at[recv_v.at[j]], dsem, add=True))
        sd[0].wait()
        sd[1].wait()
        for dsc in dd:
            dsc.wait()
        return carry

    lax.fori_loop(0, nb_per_w // CH_B, chunk, 0)
    plsc.subcore_barrier()

    pltpu.sync_copy(sum_sh.at[pl.ds(s * rt, rt), :],
                    sum_out.at[c, pl.ds(s * rt, rt), :])
    pltpu.sync_copy(deg_sh.at[pl.ds(s * rt, rt), :],
                    deg_out.at[c, pl.ds(s * rt, rt), :])


def _sc_aggregate(x, send2d, recv2d, r_rows):
    d = x.shape[1]
    zrow = jnp.zeros((ZCH, d), jnp.float32)
    zdeg = jnp.zeros((ZCH_D, DEG_L), jnp.float32)
    ones = jnp.ones((BATCH, DEG_L), jnp.float32)

    mesh = plsc.VectorSubcoreMesh(core_axis_name="c", subcore_axis_name="s")
    f = pl.kernel(
        _sc_agg_body,
        out_type=(
            jax.ShapeDtypeStruct((NC, r_rows, d), jnp.float32),
            jax.ShapeDtypeStruct((NC, r_rows, DEG_L), jnp.float32),
        ),
        mesh=mesh,
        scratch_types=(
            pltpu.VMEM((CH_B, BATCH), jnp.int32),
            pltpu.VMEM((CH_B, BATCH), jnp.int32),
            pltpu.VMEM((BATCH, d), jnp.float32),
            pltpu.VMEM((BATCH, d), jnp.float32),
            pltpu.VMEM((BATCH, DEG_L), jnp.float32),
            pltpu.VMEM((ZCH, d), jnp.float32),
            pltpu.VMEM((ZCH_D, DEG_L), jnp.float32),
            pltpu.VMEM_SHARED((r_rows, d), jnp.float32),
            pltpu.VMEM_SHARED((r_rows, DEG_L), jnp.float32),
            pltpu.SemaphoreType.DMA,
            pltpu.SemaphoreType.DMA,
            pltpu.SemaphoreType.DMA,
            pltpu.SemaphoreType.DMA,
            pltpu.SemaphoreType.DMA,
        ),
        compiler_params=pltpu.CompilerParams(use_tc_tiling_on_sc=False),
    )
    return f(x, send2d, recv2d, zrow, zdeg, ones)


def _finalize_body(x_ref, sum_ref, deg_ref, wsT_ref, wnT_ref, b_ref,
                   scale_ref, out_ref):
    xb = x_ref[...]
    sb = sum_ref[0] + sum_ref[1]
    db = deg_ref[0, :, 0:1] + deg_ref[1, :, 0:1]
    recip = jnp.where(db > 0, scale_ref[0] / db, 0.0)
    agg = sb * recip
    acc = lax.dot_general(xb, wsT_ref[...], (((1,), (0,)), ((), ())),
                          precision=lax.Precision.HIGHEST,
                          preferred_element_type=jnp.float32)
    acc = acc + lax.dot_general(agg, wnT_ref[...], (((1,), (0,)), ((), ())),
                                precision=lax.Precision.HIGHEST,
                                preferred_element_type=jnp.float32)
    out_ref[...] = acc + b_ref[...]


def _finalize(x, sum_p, deg_p, w_self, b_self, w_neigh, scale):
    n, d = x.shape
    bn = 1000
    grid = (n // bn,)
    return pl.pallas_call(
        _finalize_body,
        grid=grid,
        in_specs=[
            pl.BlockSpec((bn, d), lambda i: (i, 0)),
            pl.BlockSpec((NC, bn, d), lambda i: (0, i, 0)),
            pl.BlockSpec((NC, bn, DEG_L), lambda i: (0, i, 0)),
            pl.BlockSpec((d, d), lambda i: (0, 0)),
            pl.BlockSpec((d, d), lambda i: (0, 0)),
            pl.BlockSpec((1, d), lambda i: (0, 0)),
            pl.BlockSpec(memory_space=pltpu.SMEM),
        ],
        out_specs=pl.BlockSpec((bn, d), lambda i: (i, 0)),
        out_shape=jax.ShapeDtypeStruct((n, d), jnp.float32),
    )(x, sum_p, deg_p, w_self.T, w_neigh.T, b_self.reshape(1, d), scale)


def kernel(x, senders, receivers, w_self, b_self, w_neigh, zero_edges):
    n, d = x.shape
    e = senders.shape[0]
    senders = senders.astype(jnp.int32)
    receivers = receivers.astype(jnp.int32)

    nb_total = -(-(-(-e // BATCH)) // (NW * CH_B)) * NW * CH_B
    pad_e = nb_total * BATCH - e
    rt = -(-(n + 1) // (NW * ZCH_D)) * ZCH_D
    r_rows = NW * rt
    if pad_e:
        senders = jnp.concatenate(
            [senders, jnp.zeros((pad_e,), jnp.int32)])
        receivers = jnp.concatenate(
            [receivers,
             n + (jnp.arange(pad_e, dtype=jnp.int32) % (r_rows - n))])
    send2d = senders.reshape(nb_total, BATCH)
    recv2d = receivers.reshape(nb_total, BATCH)

    sum_p, deg_p = _sc_aggregate(x, send2d, recv2d, r_rows)

    scale = jnp.where(zero_edges, 0.0, 1.0).astype(jnp.float32).reshape(1)
    return _finalize(x, sum_p, deg_p, w_self, b_self, w_neigh, scale)
